# baseline (speedup 1.0000x reference)
import functools

import jax
import jax.numpy as jnp
from jax import lax
from jax.experimental import pallas as pl
from jax.experimental.pallas import tpu as pltpu
from jax.experimental.pallas import tpu_sc as plsc

N = 10000
E = 320000
D_IN = 128
D_HID = 64
D_OUT = 128

NPAD = 10240
NW = 32
CHUNK = 125
CHUNKS_PER_W = 80
EPW = CHUNK * CHUNKS_PER_W
ZERO_PER_SUB = N // 16
DRAIN_PER_SUB = NPAD // 16

_mesh = plsc.VectorSubcoreMesh(core_axis_name="c", subcore_axis_name="s")
_sc_params = pltpu.CompilerParams(use_tc_tiling_on_sc=False)


def _zero_rows(buf, d):
    z16 = jnp.zeros((16,), jnp.float32)

    @pl.loop(0, CHUNK)
    def _(i):
        for k in range(d // 16):
            buf[i, pl.ds(k * 16, 16)] = z16


def _zero_acc(acc_sh, buf, s):
    per = ZERO_PER_SUB // CHUNK

    @pl.loop(0, per)
    def _(t):
        pltpu.sync_copy(buf, acc_sh.at[pl.ds((s * per + t) * CHUNK, CHUNK)])


def _drain(acc_sh, out_hbm, c, s):
    pltpu.sync_copy(acc_sh.at[pl.ds(s * DRAIN_PER_SUB, DRAIN_PER_SUB)],
                    out_hbm.at[c, pl.ds(s * DRAIN_PER_SUB, DRAIN_PER_SUB)])


@jax.jit
def _sc_deg(dst3):
    @functools.partial(
        pl.kernel,
        out_type=jax.ShapeDtypeStruct((2, NPAD, 16), jnp.float32),
        mesh=_mesh,
        compiler_params=_sc_params,
        scratch_types=[
            pltpu.VMEM((CHUNKS_PER_W, CHUNK), jnp.int32),
            pltpu.VMEM((CHUNK, 16), jnp.float32),
            pltpu.VMEM_SHARED((NPAD, 16), jnp.float32),
            pltpu.SemaphoreType.DMA,
            pltpu.SemaphoreType.DMA,
        ],
    )
    def k(dst_hbm, out_hbm, dst_v, ones_v, acc_sh, s0, s1):
        c = lax.axis_index("c")
        s = lax.axis_index("s")
        wid = s * 2 + c
        pltpu.sync_copy(dst_hbm.at[wid], dst_v)
        _zero_rows(ones_v, 16)
        _zero_acc(acc_sh, ones_v, s)
        o16 = jnp.ones((16,), jnp.float32)

        @pl.loop(0, CHUNK)
        def _(i):
            ones_v[i, pl.ds(0, 16)] = o16

        plsc.subcore_barrier()

        sems = (s0, s1)

        def scat_start(j, b):
            pltpu.async_copy(ones_v, acc_sh.at[dst_v.at[j]], sems[b], add=True)

        def scat_wait(j, b):
            pltpu.make_async_copy(ones_v, acc_sh.at[dst_v.at[j]], sems[b]).wait()

        scat_start(0, 0)
        scat_start(1, 1)

        @pl.loop(0, CHUNKS_PER_W - 2, step=2)
        def _(j):
            scat_wait(j, 0)
            scat_start(j + 2, 0)
            scat_wait(j + 1, 1)
            scat_start(j + 3, 1)

        scat_wait(CHUNKS_PER_W - 2, 0)
        scat_wait(CHUNKS_PER_W - 1, 1)
        plsc.subcore_barrier()
        _drain(acc_sh, out_hbm, c, s)

    return k(dst3)


@jax.jit
def _sc_conv64(g, src3, dst3):
    d = D_HID

    @functools.partial(
        pl.kernel,
        out_type=jax.ShapeDtypeStruct((2, NPAD, d), jnp.float32),
        mesh=_mesh,
        compiler_params=_sc_params,
        scratch_types=[
            pltpu.VMEM((CHUNKS_PER_W, CHUNK), jnp.int32),
            pltpu.VMEM((CHUNKS_PER_W, CHUNK), jnp.int32),
        ] + [pltpu.VMEM((CHUNK, d), jnp.float32)] * 4
          + [pltpu.SemaphoreType.DMA] * 8
          + [pltpu.VMEM_SHARED((NPAD, d), jnp.float32)],
    )
    def k(g_hbm, src_hbm, dst_hbm, out_hbm, src_v, dst_v,
          r0, r1, r2, r3, gs0, gs1, gs2, gs3, ss0, ss1, ss2, ss3, acc_sh):
        c = lax.axis_index("c")
        s = lax.axis_index("s")
        wid = s * 2 + c
        pltpu.sync_copy(src_hbm.at[wid], src_v)
        pltpu.sync_copy(dst_hbm.at[wid], dst_v)
        _zero_rows(r0, d)
        _zero_acc(acc_sh, r0, s)
        plsc.subcore_barrier()

        rows = (r0, r1, r2, r3)
        gsems = (gs0, gs1, gs2, gs3)
        ssems = (ss0, ss1, ss2, ss3)
        NB = 4

        def gather_start(j, b):
            pltpu.make_async_copy(g_hbm.at[src_v.at[j]], rows[b], gsems[b]).start()

        def gather_wait(j, b):
            pltpu.make_async_copy(g_hbm.at[src_v.at[j]], rows[b], gsems[b]).wait()

        def scatter_start(j, b):
            pltpu.async_copy(rows[b], acc_sh.at[dst_v.at[j]], ssems[b], add=True)

        def scatter_wait(j, b):
            pltpu.make_async_copy(rows[b], acc_sh.at[dst_v.at[j]], ssems[b]).wait()

        for b in range(NB):
            gather_start(b, b)

        @pl.loop(0, CHUNKS_PER_W, step=NB)
        def _(j):
            for b in range(NB):
                ch = j + b
                gather_wait(ch, b)
                scatter_start(ch, b)

                @pl.when(ch + NB < CHUNKS_PER_W)
                def _():
                    scatter_wait(ch, b)
                    gather_start(ch + NB, b)

        for b in range(NB):
            scatter_wait(CHUNKS_PER_W - NB + b, b)

        plsc.subcore_barrier()
        _drain(acc_sh, out_hbm, c, s)

    return k(g, src3, dst3)


@jax.jit
def _sc_conv2ab(g2s, src3, dst3):
    d = D_HID

    @functools.partial(
        pl.kernel,
        out_type=jax.ShapeDtypeStruct((2, NPAD, d), jnp.float32),
        mesh=_mesh,
        compiler_params=_sc_params,
        scratch_types=[
            pltpu.VMEM((2, CHUNKS_PER_W, CHUNK), jnp.int32),
            pltpu.VMEM((2, CHUNKS_PER_W, CHUNK), jnp.int32),
        ] + [pltpu.VMEM((CHUNK, d), jnp.float32)] * 4
          + [pltpu.SemaphoreType.DMA] * 8
          + [pltpu.VMEM_SHARED((NPAD, d), jnp.float32)],
    )
    def k(g_hbm, src_hbm, dst_hbm, out_hbm, src_v, dst_v,
          r0, r1, r2, r3, gs0, gs1, gs2, gs3, ss0, ss1, ss2, ss3, acc_sh):
        c = lax.axis_index("c")
        s = lax.axis_index("s")
        pltpu.sync_copy(src_hbm.at[pl.ds(2 * s, 2)], src_v)
        pltpu.sync_copy(dst_hbm.at[pl.ds(2 * s, 2)], dst_v)
        _zero_rows(r0, d)
        _zero_acc(acc_sh, r0, s)
        plsc.subcore_barrier()

        rows = (r0, r1, r2, r3)
        gsems = (gs0, gs1, gs2, gs3)
        ssems = (ss0, ss1, ss2, ss3)
        NB = 4
        gc = g_hbm.at[c]

        for w in range(2):
            def gather_start(j, b, w=w):
                pltpu.make_async_copy(gc.at[src_v.at[w, j]], rows[b], gsems[b]).start()

            def gather_wait(j, b, w=w):
                pltpu.make_async_copy(gc.at[src_v.at[w, j]], rows[b], gsems[b]).wait()

            def scatter_start(j, b, w=w):
                pltpu.async_copy(rows[b], acc_sh.at[dst_v.at[w, j]], ssems[b], add=True)

            def scatter_wait(j, b, w=w):
                pltpu.make_async_copy(rows[b], acc_sh.at[dst_v.at[w, j]], ssems[b]).wait()

            for b in range(NB):
                gather_start(b, b)

            @pl.loop(0, CHUNKS_PER_W, step=NB)
            def _(j):
                for b in range(NB):
                    ch = j + b
                    gather_wait(ch, b)
                    scatter_start(ch, b)

                    @pl.when(ch + NB < CHUNKS_PER_W)
                    def _():
                        scatter_wait(ch, b)
                        gather_start(ch + NB, b)

            for b in range(NB):
                scatter_wait(CHUNKS_PER_W - NB + b, b)

        plsc.subcore_barrier()
        _drain(acc_sh, out_hbm, c, s)

    return k(g2s, src3, dst3)


_BLK = 512
_GRID = NPAD // _BLK


def _rowspec(d):
    return pl.BlockSpec((_BLK, d), lambda i: (i, 0))


def _accspec(d):
    return pl.BlockSpec((2, _BLK, d), lambda i: (0, i, 0))


def _fullspec(shape):
    return pl.BlockSpec(shape, lambda i: tuple(0 for _ in shape))


def _dis_of(hp):
    deg = 1.0 + hp[0, :, 0:1] + hp[1, :, 0:1]
    return lax.rsqrt(deg)


@jax.jit
def _tc_mm1(xp, W1):
    def body(x_ref, w_ref, o_ref):
        o_ref[...] = jnp.dot(x_ref[...], w_ref[...], preferred_element_type=jnp.float32)

    return pl.pallas_call(
        body,
        grid=(_GRID,),
        in_specs=[_rowspec(D_IN), _fullspec((D_IN, D_HID))],
        out_specs=_rowspec(D_HID),
        out_shape=jax.ShapeDtypeStruct((NPAD, D_HID), jnp.float32),
    )(xp, W1)


@jax.jit
def _tc_scale(h1, hist):
    def body(h_ref, hist_ref, o_ref):
        o_ref[...] = h_ref[...] * _dis_of(hist_ref[...])

    return pl.pallas_call(
        body,
        grid=(_GRID,),
        in_specs=[_rowspec(D_HID), _accspec(16)],
        out_specs=_rowspec(D_HID),
        out_shape=jax.ShapeDtypeStruct((NPAD, D_HID), jnp.float32),
    )(h1, hist)


@jax.jit
def _tc_mid(acc1, g1, hist, xp, b1, Wl, bl, W2, b2):
    def body(acc_ref, g1_ref, hist_ref, x_ref,
             b1_ref, wl_ref, bl_ref, w2_ref, b2_ref, g2s_ref, base_ref):
        dis = _dis_of(hist_ref[...])
        h = dis * (acc_ref[0] + acc_ref[1] + g1_ref[...]) + b1_ref[...][None, :]
        x1 = jnp.dot(h, wl_ref[...], preferred_element_type=jnp.float32) + bl_ref[...][None, :]
        hr = jnp.maximum(h, 0.0)
        g2 = jnp.dot(hr, w2_ref[...], preferred_element_type=jnp.float32) * dis
        g2s_ref[0] = g2[:, :D_HID]
        g2s_ref[1] = g2[:, D_HID:]
        base_ref[...] = x_ref[...] + x1 + b2_ref[...][None, :]

    return pl.pallas_call(
        body,
        grid=(_GRID,),
        in_specs=[
            _accspec(D_HID), _rowspec(D_HID),
            _accspec(16), _rowspec(D_IN),
            _fullspec((D_HID,)), _fullspec((D_HID, D_OUT)),
            _fullspec((D_OUT,)), _fullspec((D_HID, D_OUT)),
            _fullspec((D_OUT,)),
        ],
        out_specs=[_accspec(D_HID), _rowspec(D_OUT)],
        out_shape=[
            jax.ShapeDtypeStruct((2, NPAD, D_HID), jnp.float32),
            jax.ShapeDtypeStruct((NPAD, D_OUT), jnp.float32),
        ],
    )(acc1, g1, hist, xp, b1, Wl, bl, W2, b2)


@jax.jit
def _tc_final(acc2, g2s, hist, base):
    def body(acc_ref, g2s_ref, hist_ref, base_ref, o_ref):
        dis = _dis_of(hist_ref[...])
        lo = dis * (acc_ref[0] + g2s_ref[0])
        hi = dis * (acc_ref[1] + g2s_ref[1])
        o_ref[...] = jnp.concatenate([lo, hi], axis=1) + base_ref[...]

    return pl.pallas_call(
        body,
        grid=(_GRID,),
        in_specs=[
            _accspec(D_HID), _accspec(D_HID),
            _accspec(16), _rowspec(D_OUT),
        ],
        out_specs=_rowspec(D_OUT),
        out_shape=jax.ShapeDtypeStruct((NPAD, D_OUT), jnp.float32),
    )(acc2, g2s, hist, base)


def kernel(x, edge_index, W1, b1, Wl, bl, W2, b2):
    xp = jnp.pad(x, ((0, NPAD - N), (0, 0)))
    src3 = edge_index[0].reshape(NW, CHUNKS_PER_W, CHUNK)
    dst3 = edge_index[1].reshape(NW, CHUNKS_PER_W, CHUNK)

    h1 = _tc_mm1(xp, W1)
    hist = _sc_deg(dst3)
    g1 = _tc_scale(h1, hist)
    acc1 = _sc_conv64(g1, src3, dst3)
    g2s, base = _tc_mid(acc1, g1, hist, xp, b1, Wl, bl, W2, b2)
    acc2 = _sc_conv2ab(g2s, src3, dst3)
    outp = _tc_final(acc2, g2s, hist, base)
    return outp[:N]


# speedup vs baseline: 1.1070x; 1.1070x over previous
import functools

import jax
import jax.numpy as jnp
from jax import lax
from jax.experimental import pallas as pl
from jax.experimental.pallas import tpu as pltpu
from jax.experimental.pallas import tpu_sc as plsc

N = 10000
E = 320000
D_IN = 128
D_HID = 64
D_OUT = 128

NPAD = 10240
NW = 32
CHUNK = 125
CHUNKS_PER_W = 80
EPW = CHUNK * CHUNKS_PER_W
ZERO_PER_SUB = N // 16
DRAIN_PER_SUB = NPAD // 16

_mesh = plsc.VectorSubcoreMesh(core_axis_name="c", subcore_axis_name="s")
_sc_params = pltpu.CompilerParams(use_tc_tiling_on_sc=False)


def _zero_rows(buf, d):
    z16 = jnp.zeros((16,), jnp.float32)

    @pl.loop(0, CHUNK)
    def _(i):
        for k in range(d // 16):
            buf[i, pl.ds(k * 16, 16)] = z16


def _zero_acc(acc_sh, buf, s):
    per = ZERO_PER_SUB // CHUNK

    @pl.loop(0, per)
    def _(t):
        pltpu.sync_copy(buf, acc_sh.at[pl.ds((s * per + t) * CHUNK, CHUNK)])


def _drain(acc_sh, out_hbm, c, s):
    pltpu.sync_copy(acc_sh.at[pl.ds(s * DRAIN_PER_SUB, DRAIN_PER_SUB)],
                    out_hbm.at[c, pl.ds(s * DRAIN_PER_SUB, DRAIN_PER_SUB)])


@jax.jit
def _sc_deg(dst3):
    @functools.partial(
        pl.kernel,
        out_type=jax.ShapeDtypeStruct((2, NPAD, 16), jnp.float32),
        mesh=_mesh,
        compiler_params=_sc_params,
        scratch_types=[
            pltpu.VMEM((CHUNKS_PER_W, CHUNK), jnp.int32),
            pltpu.VMEM((CHUNK, 16), jnp.float32),
            pltpu.VMEM_SHARED((NPAD, 16), jnp.float32),
            pltpu.SemaphoreType.DMA,
            pltpu.SemaphoreType.DMA,
        ],
    )
    def k(dst_hbm, out_hbm, dst_v, ones_v, acc_sh, s0, s1):
        c = lax.axis_index("c")
        s = lax.axis_index("s")
        wid = s * 2 + c
        pltpu.sync_copy(dst_hbm.at[wid], dst_v)
        _zero_rows(ones_v, 16)
        _zero_acc(acc_sh, ones_v, s)
        o16 = jnp.ones((16,), jnp.float32)

        @pl.loop(0, CHUNK)
        def _(i):
            ones_v[i, pl.ds(0, 16)] = o16

        plsc.subcore_barrier()

        sems = (s0, s1)

        def scat_start(j, b):
            pltpu.async_copy(ones_v, acc_sh.at[dst_v.at[j]], sems[b], add=True)

        def scat_wait(j, b):
            pltpu.make_async_copy(ones_v, acc_sh.at[dst_v.at[j]], sems[b]).wait()

        scat_start(0, 0)
        scat_start(1, 1)

        @pl.loop(0, CHUNKS_PER_W - 2, step=2)
        def _(j):
            scat_wait(j, 0)
            scat_start(j + 2, 0)
            scat_wait(j + 1, 1)
            scat_start(j + 3, 1)

        scat_wait(CHUNKS_PER_W - 2, 0)
        scat_wait(CHUNKS_PER_W - 1, 1)
        plsc.subcore_barrier()
        _drain(acc_sh, out_hbm, c, s)

    return k(dst3)


@jax.jit
def _sc_conv64(g, src3, dst3):
    d = D_HID

    @functools.partial(
        pl.kernel,
        out_type=jax.ShapeDtypeStruct((2, NPAD, d), jnp.float32),
        mesh=_mesh,
        compiler_params=_sc_params,
        scratch_types=[
            pltpu.VMEM((CHUNKS_PER_W, CHUNK), jnp.int32),
            pltpu.VMEM((CHUNKS_PER_W, CHUNK), jnp.int32),
        ] + [pltpu.VMEM((CHUNK, d), jnp.float32)] * 4
          + [pltpu.SemaphoreType.DMA] * 8
          + [pltpu.VMEM_SHARED((NPAD, d), jnp.float32)],
    )
    def k(g_hbm, src_hbm, dst_hbm, out_hbm, src_v, dst_v,
          r0, r1, r2, r3, gs0, gs1, gs2, gs3, ss0, ss1, ss2, ss3, acc_sh):
        c = lax.axis_index("c")
        s = lax.axis_index("s")
        wid = s * 2 + c
        pltpu.sync_copy(src_hbm.at[wid], src_v)
        pltpu.sync_copy(dst_hbm.at[wid], dst_v)
        _zero_rows(r0, d)
        _zero_acc(acc_sh, r0, s)
        plsc.subcore_barrier()

        rows = (r0, r1, r2, r3)
        gsems = (gs0, gs1, gs2, gs3)
        ssems = (ss0, ss1, ss2, ss3)
        NB = 4

        def gather_start(j, b):
            pltpu.make_async_copy(g_hbm.at[src_v.at[j]], rows[b], gsems[b]).start()

        def gather_wait(j, b):
            pltpu.make_async_copy(g_hbm.at[src_v.at[j]], rows[b], gsems[b]).wait()

        def scatter_start(j, b):
            pltpu.async_copy(rows[b], acc_sh.at[dst_v.at[j]], ssems[b], add=True)

        def scatter_wait(j, b):
            pltpu.make_async_copy(rows[b], acc_sh.at[dst_v.at[j]], ssems[b]).wait()

        for b in range(NB):
            gather_start(b, b)

        @pl.loop(0, CHUNKS_PER_W, step=NB)
        def _(j):
            for b in range(NB):
                ch = j + b
                gather_wait(ch, b)
                scatter_start(ch, b)

                @pl.when(ch + NB < CHUNKS_PER_W)
                def _():
                    scatter_wait(ch, b)
                    gather_start(ch + NB, b)

        for b in range(NB):
            scatter_wait(CHUNKS_PER_W - NB + b, b)

        plsc.subcore_barrier()
        _drain(acc_sh, out_hbm, c, s)

    return k(g, src3, dst3)


@jax.jit
def _sc_conv2ab(g2s, src3, dst3):
    d = D_HID

    @functools.partial(
        pl.kernel,
        out_type=jax.ShapeDtypeStruct((2, NPAD, d), jnp.float32),
        mesh=_mesh,
        compiler_params=_sc_params,
        scratch_types=[
            pltpu.VMEM((2, CHUNKS_PER_W, CHUNK), jnp.int32),
            pltpu.VMEM((2, CHUNKS_PER_W, CHUNK), jnp.int32),
        ] + [pltpu.VMEM((CHUNK, d), jnp.float32)] * 4
          + [pltpu.SemaphoreType.DMA] * 8
          + [pltpu.VMEM_SHARED((NPAD, d), jnp.float32)],
    )
    def k(g_hbm, src_hbm, dst_hbm, out_hbm, src_v, dst_v,
          r0, r1, r2, r3, gs0, gs1, gs2, gs3, ss0, ss1, ss2, ss3, acc_sh):
        c = lax.axis_index("c")
        s = lax.axis_index("s")
        pltpu.sync_copy(src_hbm.at[pl.ds(2 * s, 2)], src_v)
        pltpu.sync_copy(dst_hbm.at[pl.ds(2 * s, 2)], dst_v)
        _zero_rows(r0, d)
        _zero_acc(acc_sh, r0, s)
        plsc.subcore_barrier()

        rows = (r0, r1, r2, r3)
        gsems = (gs0, gs1, gs2, gs3)
        ssems = (ss0, ss1, ss2, ss3)
        NB = 4
        gc = g_hbm.at[c]

        for w in range(2):
            def gather_start(j, b, w=w):
                pltpu.make_async_copy(gc.at[src_v.at[w, j]], rows[b], gsems[b]).start()

            def gather_wait(j, b, w=w):
                pltpu.make_async_copy(gc.at[src_v.at[w, j]], rows[b], gsems[b]).wait()

            def scatter_start(j, b, w=w):
                pltpu.async_copy(rows[b], acc_sh.at[dst_v.at[w, j]], ssems[b], add=True)

            def scatter_wait(j, b, w=w):
                pltpu.make_async_copy(rows[b], acc_sh.at[dst_v.at[w, j]], ssems[b]).wait()

            for b in range(NB):
                gather_start(b, b)

            @pl.loop(0, CHUNKS_PER_W, step=NB)
            def _(j):
                for b in range(NB):
                    ch = j + b
                    gather_wait(ch, b)
                    scatter_start(ch, b)

                    @pl.when(ch + NB < CHUNKS_PER_W)
                    def _():
                        scatter_wait(ch, b)
                        gather_start(ch + NB, b)

            for b in range(NB):
                scatter_wait(CHUNKS_PER_W - NB + b, b)

        plsc.subcore_barrier()
        _drain(acc_sh, out_hbm, c, s)

    return k(g2s, src3, dst3)


_BLK = 2048
_GRID = NPAD // _BLK


def _rowspec(d):
    return pl.BlockSpec((_BLK, d), lambda i: (i, 0))


def _accspec(d):
    return pl.BlockSpec((2, _BLK, d), lambda i: (0, i, 0))


def _fullspec(shape):
    return pl.BlockSpec(shape, lambda i: tuple(0 for _ in shape))


def _dis_of(hp):
    deg = 1.0 + hp[0, :, 0:1] + hp[1, :, 0:1]
    return lax.rsqrt(deg)


@jax.jit
def _tc_mm1(xp, W1):
    def body(x_ref, w_ref, o_ref):
        o_ref[...] = jnp.dot(x_ref[...], w_ref[...], preferred_element_type=jnp.float32)

    return pl.pallas_call(
        body,
        grid=(_GRID,),
        in_specs=[_rowspec(D_IN), _fullspec((D_IN, D_HID))],
        out_specs=_rowspec(D_HID),
        out_shape=jax.ShapeDtypeStruct((NPAD, D_HID), jnp.float32),
    )(xp, W1)


@jax.jit
def _tc_scale(h1, hist):
    def body(h_ref, hist_ref, o_ref):
        o_ref[...] = h_ref[...] * _dis_of(hist_ref[...])

    return pl.pallas_call(
        body,
        grid=(_GRID,),
        in_specs=[_rowspec(D_HID), _accspec(16)],
        out_specs=_rowspec(D_HID),
        out_shape=jax.ShapeDtypeStruct((NPAD, D_HID), jnp.float32),
    )(h1, hist)


@jax.jit
def _tc_mid(acc1, g1, hist, xp, b1, Wl, bl, W2, b2):
    def body(acc_ref, g1_ref, hist_ref, x_ref,
             b1_ref, wl_ref, bl_ref, w2_ref, b2_ref, g2s_ref, base_ref):
        dis = _dis_of(hist_ref[...])
        h = dis * (acc_ref[0] + acc_ref[1] + g1_ref[...]) + b1_ref[...][None, :]
        x1 = jnp.dot(h, wl_ref[...], preferred_element_type=jnp.float32) + bl_ref[...][None, :]
        hr = jnp.maximum(h, 0.0)
        g2 = jnp.dot(hr, w2_ref[...], preferred_element_type=jnp.float32) * dis
        g2s_ref[0] = g2[:, :D_HID]
        g2s_ref[1] = g2[:, D_HID:]
        base_ref[...] = x_ref[...] + x1 + b2_ref[...][None, :]

    return pl.pallas_call(
        body,
        grid=(_GRID,),
        in_specs=[
            _accspec(D_HID), _rowspec(D_HID),
            _accspec(16), _rowspec(D_IN),
            _fullspec((D_HID,)), _fullspec((D_HID, D_OUT)),
            _fullspec((D_OUT,)), _fullspec((D_HID, D_OUT)),
            _fullspec((D_OUT,)),
        ],
        out_specs=[_accspec(D_HID), _rowspec(D_OUT)],
        out_shape=[
            jax.ShapeDtypeStruct((2, NPAD, D_HID), jnp.float32),
            jax.ShapeDtypeStruct((NPAD, D_OUT), jnp.float32),
        ],
    )(acc1, g1, hist, xp, b1, Wl, bl, W2, b2)


@jax.jit
def _tc_final(acc2, g2s, hist, base):
    def body(acc_ref, g2s_ref, hist_ref, base_ref, o_ref):
        dis = _dis_of(hist_ref[...])
        lo = dis * (acc_ref[0] + g2s_ref[0])
        hi = dis * (acc_ref[1] + g2s_ref[1])
        o_ref[...] = jnp.concatenate([lo, hi], axis=1) + base_ref[...]

    return pl.pallas_call(
        body,
        grid=(_GRID,),
        in_specs=[
            _accspec(D_HID), _accspec(D_HID),
            _accspec(16), _rowspec(D_OUT),
        ],
        out_specs=_rowspec(D_OUT),
        out_shape=jax.ShapeDtypeStruct((NPAD, D_OUT), jnp.float32),
    )(acc2, g2s, hist, base)


def kernel(x, edge_index, W1, b1, Wl, bl, W2, b2):
    xp = jnp.pad(x, ((0, NPAD - N), (0, 0)))
    src3 = edge_index[0].reshape(NW, CHUNKS_PER_W, CHUNK)
    dst3 = edge_index[1].reshape(NW, CHUNKS_PER_W, CHUNK)

    h1 = _tc_mm1(xp, W1)
    hist = _sc_deg(dst3)
    g1 = _tc_scale(h1, hist)
    acc1 = _sc_conv64(g1, src3, dst3)
    g2s, base = _tc_mid(acc1, g1, hist, xp, b1, Wl, bl, W2, b2)
    acc2 = _sc_conv2ab(g2s, src3, dst3)
    outp = _tc_final(acc2, g2s, hist, base)
    return outp[:N]


# speedup vs baseline: 1.1167x; 1.0088x over previous
import functools

import jax
import jax.numpy as jnp
from jax import lax
from jax.experimental import pallas as pl
from jax.experimental.pallas import tpu as pltpu
from jax.experimental.pallas import tpu_sc as plsc

N = 10000
E = 320000
D_IN = 128
D_HID = 64
D_OUT = 128

NPAD = 10240
NW = 32
CHUNK = 125
CHUNKS_PER_W = 80
EPW = CHUNK * CHUNKS_PER_W
ZERO_PER_SUB = N // 16
DRAIN_PER_SUB = NPAD // 16

_mesh = plsc.VectorSubcoreMesh(core_axis_name="c", subcore_axis_name="s")
_sc_params = pltpu.CompilerParams(use_tc_tiling_on_sc=False)


def _zero_rows(buf, d):
    z16 = jnp.zeros((16,), jnp.float32)

    @pl.loop(0, CHUNK)
    def _(i):
        for k in range(d // 16):
            buf[i, pl.ds(k * 16, 16)] = z16


def _zero_acc(acc_sh, buf, s):
    per = ZERO_PER_SUB // CHUNK

    @pl.loop(0, per)
    def _(t):
        pltpu.sync_copy(buf, acc_sh.at[pl.ds((s * per + t) * CHUNK, CHUNK)])


def _drain(acc_sh, out_hbm, c, s):
    pltpu.sync_copy(acc_sh.at[pl.ds(s * DRAIN_PER_SUB, DRAIN_PER_SUB)],
                    out_hbm.at[c, pl.ds(s * DRAIN_PER_SUB, DRAIN_PER_SUB)])


@jax.jit
def _sc_deg(dst3):
    @functools.partial(
        pl.kernel,
        out_type=jax.ShapeDtypeStruct((2, NPAD, 16), jnp.float32),
        mesh=_mesh,
        compiler_params=_sc_params,
        scratch_types=[
            pltpu.VMEM((CHUNKS_PER_W, CHUNK), jnp.int32),
            pltpu.VMEM((CHUNK, 16), jnp.float32),
            pltpu.VMEM_SHARED((NPAD, 16), jnp.float32),
            pltpu.SemaphoreType.DMA,
            pltpu.SemaphoreType.DMA,
        ],
    )
    def k(dst_hbm, out_hbm, dst_v, ones_v, acc_sh, s0, s1):
        c = lax.axis_index("c")
        s = lax.axis_index("s")
        wid = s * 2 + c
        pltpu.sync_copy(dst_hbm.at[wid], dst_v)
        _zero_rows(ones_v, 16)
        _zero_acc(acc_sh, ones_v, s)
        o16 = jnp.ones((16,), jnp.float32)

        @pl.loop(0, CHUNK)
        def _(i):
            ones_v[i, pl.ds(0, 16)] = o16

        plsc.subcore_barrier()

        sems = (s0, s1)

        def scat_start(j, b):
            pltpu.async_copy(ones_v, acc_sh.at[dst_v.at[j]], sems[b], add=True)

        def scat_wait(j, b):
            pltpu.make_async_copy(ones_v, acc_sh.at[dst_v.at[j]], sems[b]).wait()

        scat_start(0, 0)
        scat_start(1, 1)

        @pl.loop(0, CHUNKS_PER_W - 2, step=2)
        def _(j):
            scat_wait(j, 0)
            scat_start(j + 2, 0)
            scat_wait(j + 1, 1)
            scat_start(j + 3, 1)

        scat_wait(CHUNKS_PER_W - 2, 0)
        scat_wait(CHUNKS_PER_W - 1, 1)
        plsc.subcore_barrier()
        _drain(acc_sh, out_hbm, c, s)

    return k(dst3)


@jax.jit
def _sc_conv64(g, src3, dst3):
    d = D_HID

    @functools.partial(
        pl.kernel,
        out_type=jax.ShapeDtypeStruct((2, NPAD, d), jnp.float32),
        mesh=_mesh,
        compiler_params=_sc_params,
        scratch_types=[
            pltpu.VMEM((CHUNKS_PER_W, CHUNK), jnp.int32),
            pltpu.VMEM((CHUNKS_PER_W, CHUNK), jnp.int32),
        ] + [pltpu.VMEM((CHUNK, d), jnp.float32)] * 4
          + [pltpu.SemaphoreType.DMA] * 8
          + [pltpu.VMEM_SHARED((NPAD, d), jnp.float32)],
    )
    def k(g_hbm, src_hbm, dst_hbm, out_hbm, src_v, dst_v,
          r0, r1, r2, r3, gs0, gs1, gs2, gs3, ss0, ss1, ss2, ss3, acc_sh):
        c = lax.axis_index("c")
        s = lax.axis_index("s")
        wid = s * 2 + c
        pltpu.sync_copy(src_hbm.at[wid], src_v)
        pltpu.sync_copy(dst_hbm.at[wid], dst_v)
        _zero_rows(r0, d)
        _zero_acc(acc_sh, r0, s)
        plsc.subcore_barrier()

        rows = (r0, r1, r2, r3)
        gsems = (gs0, gs1, gs2, gs3)
        ssems = (ss0, ss1, ss2, ss3)
        NB = 4

        def gather_start(j, b):
            pltpu.make_async_copy(g_hbm.at[src_v.at[j]], rows[b], gsems[b]).start()

        def gather_wait(j, b):
            pltpu.make_async_copy(g_hbm.at[src_v.at[j]], rows[b], gsems[b]).wait()

        def scatter_start(j, b):
            pltpu.async_copy(rows[b], acc_sh.at[dst_v.at[j]], ssems[b], add=True)

        def scatter_wait(j, b):
            pltpu.make_async_copy(rows[b], acc_sh.at[dst_v.at[j]], ssems[b]).wait()

        for b in range(NB):
            gather_start(b, b)

        @pl.loop(0, CHUNKS_PER_W, step=NB)
        def _(j):
            for b in range(NB):
                ch = j + b
                gather_wait(ch, b)
                scatter_start(ch, b)

                @pl.when(ch + NB < CHUNKS_PER_W)
                def _():
                    scatter_wait(ch, b)
                    gather_start(ch + NB, b)

        for b in range(NB):
            scatter_wait(CHUNKS_PER_W - NB + b, b)

        plsc.subcore_barrier()
        _drain(acc_sh, out_hbm, c, s)

    return k(g, src3, dst3)


@jax.jit
def _sc_conv2ab(g2s, src3, dst3):
    d = D_HID

    @functools.partial(
        pl.kernel,
        out_type=jax.ShapeDtypeStruct((2, NPAD, d), jnp.float32),
        mesh=_mesh,
        compiler_params=_sc_params,
        scratch_types=[
            pltpu.VMEM((2, CHUNKS_PER_W, CHUNK), jnp.int32),
            pltpu.VMEM((2, CHUNKS_PER_W, CHUNK), jnp.int32),
        ] + [pltpu.VMEM((CHUNK, d), jnp.float32)] * 4
          + [pltpu.SemaphoreType.DMA] * 8
          + [pltpu.VMEM_SHARED((NPAD, d), jnp.float32)],
    )
    def k(g_hbm, src_hbm, dst_hbm, out_hbm, src_v, dst_v,
          r0, r1, r2, r3, gs0, gs1, gs2, gs3, ss0, ss1, ss2, ss3, acc_sh):
        c = lax.axis_index("c")
        s = lax.axis_index("s")
        pltpu.sync_copy(src_hbm.at[pl.ds(2 * s, 2)], src_v)
        pltpu.sync_copy(dst_hbm.at[pl.ds(2 * s, 2)], dst_v)
        _zero_rows(r0, d)
        _zero_acc(acc_sh, r0, s)
        plsc.subcore_barrier()

        rows = (r0, r1, r2, r3)
        gsems = (gs0, gs1, gs2, gs3)
        ssems = (ss0, ss1, ss2, ss3)
        NB = 4
        gc = g_hbm.at[c]

        for w in range(2):
            def gather_start(j, b, w=w):
                pltpu.make_async_copy(gc.at[src_v.at[w, j]], rows[b], gsems[b]).start()

            def gather_wait(j, b, w=w):
                pltpu.make_async_copy(gc.at[src_v.at[w, j]], rows[b], gsems[b]).wait()

            def scatter_start(j, b, w=w):
                pltpu.async_copy(rows[b], acc_sh.at[dst_v.at[w, j]], ssems[b], add=True)

            def scatter_wait(j, b, w=w):
                pltpu.make_async_copy(rows[b], acc_sh.at[dst_v.at[w, j]], ssems[b]).wait()

            for b in range(NB):
                gather_start(b, b)

            @pl.loop(0, CHUNKS_PER_W, step=NB)
            def _(j):
                for b in range(NB):
                    ch = j + b
                    gather_wait(ch, b)
                    scatter_start(ch, b)

                    @pl.when(ch + NB < CHUNKS_PER_W)
                    def _():
                        scatter_wait(ch, b)
                        gather_start(ch + NB, b)

            for b in range(NB):
                scatter_wait(CHUNKS_PER_W - NB + b, b)

        plsc.subcore_barrier()
        _drain(acc_sh, out_hbm, c, s)

    return k(g2s, src3, dst3)


_BLK = 5120
_GRID = NPAD // _BLK


def _rowspec(d):
    return pl.BlockSpec((_BLK, d), lambda i: (i, 0))


def _accspec(d):
    return pl.BlockSpec((2, _BLK, d), lambda i: (0, i, 0))


def _fullspec(shape):
    return pl.BlockSpec(shape, lambda i: tuple(0 for _ in shape))


def _dis_of(hp):
    deg = 1.0 + hp[0, :, 0:1] + hp[1, :, 0:1]
    return lax.rsqrt(deg)


@jax.jit
def _tc_mm1(xp, W1):
    def body(x_ref, w_ref, o_ref):
        o_ref[...] = jnp.dot(x_ref[...], w_ref[...], preferred_element_type=jnp.float32)

    return pl.pallas_call(
        body,
        grid=(_GRID,),
        in_specs=[_rowspec(D_IN), _fullspec((D_IN, D_HID))],
        out_specs=_rowspec(D_HID),
        out_shape=jax.ShapeDtypeStruct((NPAD, D_HID), jnp.float32),
    )(xp, W1)


@jax.jit
def _tc_scale(h1, hist):
    def body(h_ref, hist_ref, o_ref):
        o_ref[...] = h_ref[...] * _dis_of(hist_ref[...])

    return pl.pallas_call(
        body,
        grid=(_GRID,),
        in_specs=[_rowspec(D_HID), _accspec(16)],
        out_specs=_rowspec(D_HID),
        out_shape=jax.ShapeDtypeStruct((NPAD, D_HID), jnp.float32),
    )(h1, hist)


@jax.jit
def _tc_mid(acc1, g1, hist, xp, b1, Wl, bl, W2, b2):
    def body(acc_ref, g1_ref, hist_ref, x_ref,
             b1_ref, wl_ref, bl_ref, w2_ref, b2_ref, g2s_ref, base_ref):
        dis = _dis_of(hist_ref[...])
        h = dis * (acc_ref[0] + acc_ref[1] + g1_ref[...]) + b1_ref[...][None, :]
        x1 = jnp.dot(h, wl_ref[...], preferred_element_type=jnp.float32) + bl_ref[...][None, :]
        hr = jnp.maximum(h, 0.0)
        g2 = jnp.dot(hr, w2_ref[...], preferred_element_type=jnp.float32) * dis
        g2s_ref[0] = g2[:, :D_HID]
        g2s_ref[1] = g2[:, D_HID:]
        base_ref[...] = x_ref[...] + x1 + b2_ref[...][None, :]

    return pl.pallas_call(
        body,
        grid=(_GRID,),
        in_specs=[
            _accspec(D_HID), _rowspec(D_HID),
            _accspec(16), _rowspec(D_IN),
            _fullspec((D_HID,)), _fullspec((D_HID, D_OUT)),
            _fullspec((D_OUT,)), _fullspec((D_HID, D_OUT)),
            _fullspec((D_OUT,)),
        ],
        out_specs=[_accspec(D_HID), _rowspec(D_OUT)],
        out_shape=[
            jax.ShapeDtypeStruct((2, NPAD, D_HID), jnp.float32),
            jax.ShapeDtypeStruct((NPAD, D_OUT), jnp.float32),
        ],
    )(acc1, g1, hist, xp, b1, Wl, bl, W2, b2)


@jax.jit
def _tc_final(acc2, g2s, hist, base):
    def body(acc_ref, g2s_ref, hist_ref, base_ref, o_ref):
        dis = _dis_of(hist_ref[...])
        lo = dis * (acc_ref[0] + g2s_ref[0])
        hi = dis * (acc_ref[1] + g2s_ref[1])
        o_ref[...] = jnp.concatenate([lo, hi], axis=1) + base_ref[...]

    return pl.pallas_call(
        body,
        grid=(_GRID,),
        in_specs=[
            _accspec(D_HID), _accspec(D_HID),
            _accspec(16), _rowspec(D_OUT),
        ],
        out_specs=_rowspec(D_OUT),
        out_shape=jax.ShapeDtypeStruct((NPAD, D_OUT), jnp.float32),
    )(acc2, g2s, hist, base)


def kernel(x, edge_index, W1, b1, Wl, bl, W2, b2):
    xp = jnp.pad(x, ((0, NPAD - N), (0, 0)))
    src3 = edge_index[0].reshape(NW, CHUNKS_PER_W, CHUNK)
    dst3 = edge_index[1].reshape(NW, CHUNKS_PER_W, CHUNK)

    h1 = _tc_mm1(xp, W1)
    hist = _sc_deg(dst3)
    g1 = _tc_scale(h1, hist)
    acc1 = _sc_conv64(g1, src3, dst3)
    g2s, base = _tc_mid(acc1, g1, hist, xp, b1, Wl, bl, W2, b2)
    acc2 = _sc_conv2ab(g2s, src3, dst3)
    outp = _tc_final(acc2, g2s, hist, base)
    return outp[:N]
